# baseline (speedup 1.0000x reference)
import jax
import jax.numpy as jnp
from jax import lax
from jax.experimental import pallas as pl
from jax.experimental.pallas import tpu as pltpu

_NEG = -1e30
_TOPK = 100
_C = 80
_H = 128
_W = 128


def _decode_body(hm_ref, off_ref, wh_ref, s_ref, c_ref, b_ref, kept_ref, rmax_ref):
    negcol = jnp.full((_H, 1), _NEG, jnp.float32)
    negrow = jnp.full((1, _W), _NEG, jnp.float32)

    def nms_body(c, _):
        xc = hm_ref[0, c]
        l = jnp.concatenate([xc[:, 1:], negcol], axis=1)
        r = jnp.concatenate([negcol, xc[:, : _W - 1]], axis=1)
        h3 = jnp.maximum(xc, jnp.maximum(l, r))
        u = jnp.concatenate([h3[1:, :], negrow], axis=0)
        d = jnp.concatenate([negrow, h3[: _H - 1, :]], axis=0)
        m3 = jnp.maximum(h3, jnp.maximum(u, d))
        keptc = jnp.where(xc == m3, xc, _NEG)
        kept_ref[pl.ds(c * _H, _H), :] = keptc
        rmax_ref[pl.ds(c, 1), :] = jnp.max(keptc, axis=1).reshape(1, _H)
        return 0

    lax.fori_loop(0, _C, nms_body, 0)

    flat_iota = (
        lax.broadcasted_iota(jnp.int32, (_C, _H), 0) * _H
        + lax.broadcasted_iota(jnp.int32, (_C, _H), 1)
    )
    lane = lax.broadcasted_iota(jnp.int32, (1, _W), 1)
    zeros = jnp.zeros((1, _W), jnp.float32)
    big = jnp.int32(2**30)

    def sel_body(k, carry):
        rowmax, sc, cl, ys, xs, o0, o1, w0, w1 = carry
        m = jnp.max(rowmax)
        p = jnp.min(jnp.where(rowmax >= m, flat_iota, big))
        cls = p // _H
        y = p - cls * _H
        row = kept_ref[pl.ds(p, 1), :]
        xi = jnp.min(jnp.where(row >= m, lane, big))
        newrow = jnp.where(lane == xi, _NEG, row)
        kept_ref[pl.ds(p, 1), :] = newrow
        rowmax = jnp.where(flat_iota == p, jnp.max(newrow), rowmax)
        selm = lane == xi
        og0 = jnp.sum(jnp.where(selm, off_ref[0, pl.ds(y, 1), :], zeros))
        og1 = jnp.sum(jnp.where(selm, off_ref[0, pl.ds(y + _H, 1), :], zeros))
        wg0 = jnp.sum(jnp.where(selm, wh_ref[0, pl.ds(y, 1), :], zeros))
        wg1 = jnp.sum(jnp.where(selm, wh_ref[0, pl.ds(y + _H, 1), :], zeros))
        kv = lane == k
        sc = jnp.where(kv, m, sc)
        cl = jnp.where(kv, cls.astype(jnp.float32), cl)
        ys = jnp.where(kv, y.astype(jnp.float32), ys)
        xs = jnp.where(kv, xi.astype(jnp.float32), xs)
        o0 = jnp.where(kv, og0, o0)
        o1 = jnp.where(kv, og1, o1)
        w0 = jnp.where(kv, wg0, w0)
        w1 = jnp.where(kv, wg1, w1)
        return (rowmax, sc, cl, ys, xs, o0, o1, w0, w1)

    init = (rmax_ref[...],) + tuple(
        jnp.full((1, _W), _NEG, jnp.float32) for _ in range(8)
    )
    _, sc, cl, ys, xs, o0, o1, w0, w1 = lax.fori_loop(0, _TOPK, sel_body, init)

    score = jax.nn.sigmoid(sc)
    xs = xs + o0
    ys = ys + o1
    x1 = jnp.maximum((xs - w0 * 0.5) * 4.0, 0.0)
    y1 = jnp.maximum((ys - w1 * 0.5) * 4.0, 0.0)
    x2 = jnp.minimum((xs + w0 * 0.5) * 4.0, 511.0)
    y2 = jnp.minimum((ys + w1 * 0.5) * 4.0, 511.0)
    mask = score > 0.05
    s_ref[0] = jnp.where(mask, score, -1.0)
    c_ref[0] = jnp.where(mask, cl, -1.0)
    b_ref[0] = jnp.concatenate(
        [
            jnp.where(mask, x1, -1.0),
            jnp.where(mask, y1, -1.0),
            jnp.where(mask, x2, -1.0),
            jnp.where(mask, y2, -1.0),
        ],
        axis=0,
    )


def kernel(heatmap_heads, offset_heads, wh_heads):
    B = heatmap_heads.shape[0]
    off_r = offset_heads.reshape(B, 2 * _H, _W)
    wh_r = wh_heads.reshape(B, 2 * _H, _W)
    s, c, b = pl.pallas_call(
        _decode_body,
        grid=(B,),
        in_specs=[
            pl.BlockSpec((1, _C, _H, _W), lambda i: (i, 0, 0, 0)),
            pl.BlockSpec((1, 2 * _H, _W), lambda i: (i, 0, 0)),
            pl.BlockSpec((1, 2 * _H, _W), lambda i: (i, 0, 0)),
        ],
        out_specs=[
            pl.BlockSpec((1, 1, _W), lambda i: (i, 0, 0)),
            pl.BlockSpec((1, 1, _W), lambda i: (i, 0, 0)),
            pl.BlockSpec((1, 4, _W), lambda i: (i, 0, 0)),
        ],
        out_shape=[
            jax.ShapeDtypeStruct((B, 1, _W), jnp.float32),
            jax.ShapeDtypeStruct((B, 1, _W), jnp.float32),
            jax.ShapeDtypeStruct((B, 4, _W), jnp.float32),
        ],
        scratch_shapes=[
            pltpu.VMEM((_C * _H, _W), jnp.float32),
            pltpu.VMEM((_C, _H), jnp.float32),
        ],
        compiler_params=pltpu.CompilerParams(
            dimension_semantics=("arbitrary",)
        ),
    )(heatmap_heads, off_r, wh_r)
    scores = s[:, 0, :_TOPK]
    classes = c[:, 0, :_TOPK]
    boxes = jnp.transpose(b, (0, 2, 1))[:, :_TOPK, :]
    return scores, classes, boxes
